# baseline (speedup 1.0000x reference)
import functools

import numpy as np
import jax
import jax.numpy as jnp
from jax import lax
from jax.experimental import pallas as pl
from jax.experimental.pallas import tpu as pltpu
from jax.experimental.pallas import tpu_sc as plsc

N_NODES = 10000
N_EDGES = 320000
D_NODE = 128
D_EDGE = 16
D_OUT = 128
D_W = D_OUT // 2

NC = 2
NS = 16
NW = NC * NS
CH = 128
CH2 = CH // 2
NCHUNKS = N_EDGES // CH
CH_PER_W = NCHUNKS // NW
CH_EXTRA = NCHUNKS - CH_PER_W * NW
NPAIR = CH_PER_W // 2

_LO_SEL = np.concatenate([np.arange(32 * m, 32 * m + 16) for m in range(4)])
_HI_SEL = _LO_SEL + 16


def _node_proj_body(x_ref, w_ref, b_ref, p_ref, q_ref):
    x = x_ref[...]
    w1 = w_ref[0:D_NODE, :]
    w2 = w_ref[D_NODE:2 * D_NODE, :]
    p_ref[...] = jnp.dot(x, w1, preferred_element_type=jnp.float32) + b_ref[...]
    q_ref[...] = jnp.dot(x, w2, preferred_element_type=jnp.float32)


def _node_proj(x, W, b):
    return pl.pallas_call(
        _node_proj_body,
        out_shape=(
            jax.ShapeDtypeStruct((N_NODES, D_OUT), jnp.float32),
            jax.ShapeDtypeStruct((N_NODES, D_OUT), jnp.float32),
        ),
    )(x, W, b.reshape(1, D_OUT))


_EBLK = 16000


def _bf16_bits_rounded(v):
    i = lax.bitcast_convert_type(v, jnp.int32)
    return i + np.int32(0x7FFF) + ((i >> 16) & np.int32(1))


def _edge_proj_body(ea_ref, w3p_ref, e_ref):
    e2 = jnp.dot(ea_ref[...], w3p_ref[...], preferred_element_type=jnp.float32)
    rlo = _bf16_bits_rounded(e2[:, :D_W])
    rhi = _bf16_bits_rounded(e2[:, D_W:])
    w = ((rlo >> 16) & np.int32(0xFFFF)) | (rhi & _HI_MASK)
    e_ref[...] = w.reshape(_EBLK // 2, 2, D_W)


def _edge_proj(edge_attr, W3p):
    grid = (N_EDGES // _EBLK,)
    return pl.pallas_call(
        _edge_proj_body,
        grid=grid,
        in_specs=[
            pl.BlockSpec((_EBLK, D_EDGE), lambda i: (i, 0)),
            pl.BlockSpec((D_EDGE, D_OUT), lambda i: (0, 0)),
        ],
        out_specs=pl.BlockSpec((_EBLK // 2, 2, D_W), lambda i: (i, 0, 0)),
        out_shape=jax.ShapeDtypeStruct((N_EDGES // 2, 2, D_W), jnp.int32),
    )(edge_attr, W3p)


_HI_MASK = np.int32(-65536)


def _decode(w):
    lo = lax.bitcast_convert_type(w << 16, jnp.float32)
    hi = lax.bitcast_convert_type(w & _HI_MASK, jnp.float32)
    return lo, hi


def _sc_body(src_ref, dst_ref, p_ref, q_ref, e_ref, out_ref,
             idx_s, idx_d, pbuf, qbuf, ebuf,
             sem_g0, sem_g1, sem_w0, sem_w1):
    sem_g = (sem_g0, sem_g1)
    sem_w = (sem_w0, sem_w1)
    c = lax.axis_index("c")
    s = lax.axis_index("s")
    wid = s * NC + c
    start_ck = wid * CH_PER_W + jnp.minimum(wid, CH_EXTRA)

    def issue(b, ck):
        base = ck * CH
        pltpu.sync_copy(src_ref.at[pl.ds(base, CH)], idx_s.at[b])
        pltpu.sync_copy(dst_ref.at[pl.ds(base, CH)], idx_d.at[b])
        pltpu.async_copy(p_ref.at[idx_s.at[b]], pbuf.at[b], sem_g[b])
        pltpu.async_copy(q_ref.at[idx_d.at[b]], qbuf.at[b], sem_g[b])
        pltpu.async_copy(e_ref.at[pl.ds(ck * CH2, CH2)], ebuf.at[b], sem_g[b])

    def wait_in(b, ck):
        pltpu.make_async_copy(p_ref.at[idx_s.at[b]], pbuf.at[b], sem_g[b]).wait()
        pltpu.make_async_copy(q_ref.at[idx_d.at[b]], qbuf.at[b], sem_g[b]).wait()
        pltpu.make_async_copy(
            e_ref.at[pl.ds(ck * CH2, CH2)], ebuf.at[b], sem_g[b]).wait()

    def compute(b):
        def row_body(r2, rcarry):
            for half in range(2):
                r = r2 * 2 + half
                for m in range(4):
                    ew = ebuf[b, r2, half, pl.ds(16 * m, 16)]
                    elo, ehi = _decode(ew)
                    sl_lo = pl.ds(32 * m, 16)
                    sl_hi = pl.ds(32 * m + 16, 16)
                    plsc.addupdate(pbuf.at[b, r, sl_lo], qbuf[b, r, sl_lo] + elo)
                    plsc.addupdate(pbuf.at[b, r, sl_hi], qbuf[b, r, sl_hi] + ehi)
            return rcarry
        lax.fori_loop(0, CH2, row_body, 0)

    def issue_out(b, ck):
        pltpu.async_copy(pbuf.at[b], out_ref.at[pl.ds(ck * CH, CH)], sem_w[b])

    def wait_out(b, ck):
        pltpu.make_async_copy(
            pbuf.at[b], out_ref.at[pl.ds(ck * CH, CH)], sem_w[b]).wait()

    def pair_body(i, carry):
        k0 = start_ck + 2 * i

        @pl.when(i > 0)
        def _():
            wait_out(1, k0 - 1)

        issue(1, k0 + 1)
        wait_in(0, k0)
        compute(0)
        issue_out(0, k0)
        wait_in(1, k0 + 1)
        compute(1)

        @pl.when(i < NPAIR - 1)
        def _():
            wait_out(0, k0)
            issue(0, k0 + 2)

        issue_out(1, k0 + 1)
        return carry

    issue(0, start_ck)
    lax.fori_loop(0, NPAIR, pair_body, 0)
    wait_out(0, start_ck + CH_PER_W - 2)
    wait_out(1, start_ck + CH_PER_W - 1)

    @pl.when(wid < CH_EXTRA)
    def _():
        ck = start_ck + CH_PER_W
        issue(0, ck)
        wait_in(0, ck)
        compute(0)
        issue_out(0, ck)
        wait_out(0, ck)


def _sc_gather_add(src, dst, P, Q, E):
    mesh = plsc.VectorSubcoreMesh(
        core_axis_name="c", subcore_axis_name="s", num_cores=NC, num_subcores=NS)
    k = functools.partial(
        pl.kernel,
        mesh=mesh,
        out_type=jax.ShapeDtypeStruct((N_EDGES, D_OUT), jnp.float32),
        scratch_types=[
            pltpu.VMEM((2, CH), jnp.int32),
            pltpu.VMEM((2, CH), jnp.int32),
            pltpu.VMEM((2, CH, D_OUT), jnp.float32),
            pltpu.VMEM((2, CH, D_OUT), jnp.float32),
            pltpu.VMEM((2, CH2, 2, D_W), jnp.int32),
            pltpu.SemaphoreType.DMA,
            pltpu.SemaphoreType.DMA,
            pltpu.SemaphoreType.DMA,
            pltpu.SemaphoreType.DMA,
        ],
    )(_sc_body)
    return k(src, dst, P, Q, E)


def kernel(x, edge_index, edge_attr, W, b):
    W3 = W[2 * D_NODE:, :]
    W3p = jnp.concatenate(
        [W3[:, jnp.asarray(_LO_SEL)], W3[:, jnp.asarray(_HI_SEL)]], axis=1)
    P, Q = _node_proj(x, W, b)
    Ew = _edge_proj(edge_attr, W3p)
    return _sc_gather_add(edge_index[0], edge_index[1], P, Q, Ew)


# speedup vs baseline: 1.2016x; 1.2016x over previous
import functools

import numpy as np
import jax
import jax.numpy as jnp
from jax import lax
from jax.experimental import pallas as pl
from jax.experimental.pallas import tpu as pltpu
from jax.experimental.pallas import tpu_sc as plsc

N_NODES = 10000
N_EDGES = 320000
D_NODE = 128
D_EDGE = 16
D_OUT = 128
D_W = D_OUT // 2

NC = 2
NS = 16
NW = NC * NS
CH = 128
CH2 = CH // 2
NCHUNKS = N_EDGES // CH
CH_PER_W = NCHUNKS // NW
CH_EXTRA = NCHUNKS - CH_PER_W * NW
NPAIR = CH_PER_W // 2

_LO_SEL = np.concatenate([np.arange(32 * m, 32 * m + 16) for m in range(4)])
_HI_SEL = _LO_SEL + 16


def _node_proj_body(x_ref, w_ref, b_ref, p_ref, q_ref):
    x = x_ref[...]
    w1 = w_ref[0:D_NODE, :]
    w2 = w_ref[D_NODE:2 * D_NODE, :]
    p_ref[...] = jnp.dot(x, w1, preferred_element_type=jnp.float32) + b_ref[...]
    q_ref[...] = jnp.dot(x, w2, preferred_element_type=jnp.float32)


def _node_proj(x, W, b):
    return pl.pallas_call(
        _node_proj_body,
        out_shape=(
            jax.ShapeDtypeStruct((N_NODES, D_OUT), jnp.float32),
            jax.ShapeDtypeStruct((N_NODES, D_OUT), jnp.float32),
        ),
    )(x, W, b.reshape(1, D_OUT))


_EBLK = 16000


def _bf16_bits_rounded(v):
    i = lax.bitcast_convert_type(v, jnp.int32)
    return i + np.int32(0x7FFF) + ((i >> 16) & np.int32(1))


def _edge_proj_body(eat_ref, w3p_ref, e_ref):
    e2 = lax.dot_general(
        eat_ref[...], w3p_ref[...],
        dimension_numbers=(((0,), (0,)), ((), ())),
        preferred_element_type=jnp.float32)
    rlo = _bf16_bits_rounded(e2[:, :D_W])
    rhi = _bf16_bits_rounded(e2[:, D_W:])
    w = ((rlo >> 16) & np.int32(0xFFFF)) | (rhi & _HI_MASK)
    e_ref[...] = w.reshape(_EBLK // 2, 2, D_W)


def _edge_proj(edge_attr_t, W3p):
    grid = (N_EDGES // _EBLK,)
    return pl.pallas_call(
        _edge_proj_body,
        grid=grid,
        in_specs=[
            pl.BlockSpec((D_EDGE, _EBLK), lambda i: (0, i)),
            pl.BlockSpec((D_EDGE, D_OUT), lambda i: (0, 0)),
        ],
        out_specs=pl.BlockSpec((_EBLK // 2, 2, D_W), lambda i: (i, 0, 0)),
        out_shape=jax.ShapeDtypeStruct((N_EDGES // 2, 2, D_W), jnp.int32),
    )(edge_attr_t, W3p)


_HI_MASK = np.int32(-65536)


def _decode(w):
    lo = lax.bitcast_convert_type(w << 16, jnp.float32)
    hi = lax.bitcast_convert_type(w & _HI_MASK, jnp.float32)
    return lo, hi


def _sc_body(src_ref, dst_ref, p_ref, q_ref, e_ref, out_ref,
             idx_s, idx_d, pbuf, qbuf, ebuf,
             sem_g0, sem_g1, sem_w0, sem_w1):
    sem_g = (sem_g0, sem_g1)
    sem_w = (sem_w0, sem_w1)
    c = lax.axis_index("c")
    s = lax.axis_index("s")
    wid = s * NC + c
    start_ck = wid * CH_PER_W + jnp.minimum(wid, CH_EXTRA)

    def issue(b, ck):
        base = ck * CH
        pltpu.sync_copy(src_ref.at[pl.ds(base, CH)], idx_s.at[b])
        pltpu.sync_copy(dst_ref.at[pl.ds(base, CH)], idx_d.at[b])
        pltpu.async_copy(p_ref.at[idx_s.at[b]], pbuf.at[b], sem_g[b])
        pltpu.async_copy(q_ref.at[idx_d.at[b]], qbuf.at[b], sem_g[b])
        pltpu.async_copy(e_ref.at[pl.ds(ck * CH2, CH2)], ebuf.at[b], sem_g[b])

    def wait_in(b, ck):
        pltpu.make_async_copy(p_ref.at[idx_s.at[b]], pbuf.at[b], sem_g[b]).wait()
        pltpu.make_async_copy(q_ref.at[idx_d.at[b]], qbuf.at[b], sem_g[b]).wait()
        pltpu.make_async_copy(
            e_ref.at[pl.ds(ck * CH2, CH2)], ebuf.at[b], sem_g[b]).wait()

    def compute(b):
        def row_body(r2, rcarry):
            for half in range(2):
                r = r2 * 2 + half
                for m in range(4):
                    ew = ebuf[b, r2, half, pl.ds(16 * m, 16)]
                    elo, ehi = _decode(ew)
                    sl_lo = pl.ds(32 * m, 16)
                    sl_hi = pl.ds(32 * m + 16, 16)
                    plsc.addupdate(pbuf.at[b, r, sl_lo], qbuf[b, r, sl_lo] + elo)
                    plsc.addupdate(pbuf.at[b, r, sl_hi], qbuf[b, r, sl_hi] + ehi)
            return rcarry
        lax.fori_loop(0, CH2, row_body, 0)

    def issue_out(b, ck):
        pltpu.async_copy(pbuf.at[b], out_ref.at[pl.ds(ck * CH, CH)], sem_w[b])

    def wait_out(b, ck):
        pltpu.make_async_copy(
            pbuf.at[b], out_ref.at[pl.ds(ck * CH, CH)], sem_w[b]).wait()

    def pair_body(i, carry):
        k0 = start_ck + 2 * i

        @pl.when(i > 0)
        def _():
            wait_out(1, k0 - 1)

        issue(1, k0 + 1)
        wait_in(0, k0)
        compute(0)
        issue_out(0, k0)
        wait_in(1, k0 + 1)
        compute(1)

        @pl.when(i < NPAIR - 1)
        def _():
            wait_out(0, k0)
            issue(0, k0 + 2)

        issue_out(1, k0 + 1)
        return carry

    issue(0, start_ck)
    lax.fori_loop(0, NPAIR, pair_body, 0)
    wait_out(0, start_ck + CH_PER_W - 2)
    wait_out(1, start_ck + CH_PER_W - 1)

    @pl.when(wid < CH_EXTRA)
    def _():
        ck = start_ck + CH_PER_W
        issue(0, ck)
        wait_in(0, ck)
        compute(0)
        issue_out(0, ck)
        wait_out(0, ck)


def _sc_gather_add(src, dst, P, Q, E):
    mesh = plsc.VectorSubcoreMesh(
        core_axis_name="c", subcore_axis_name="s", num_cores=NC, num_subcores=NS)
    k = functools.partial(
        pl.kernel,
        mesh=mesh,
        out_type=jax.ShapeDtypeStruct((N_EDGES, D_OUT), jnp.float32),
        scratch_types=[
            pltpu.VMEM((2, CH), jnp.int32),
            pltpu.VMEM((2, CH), jnp.int32),
            pltpu.VMEM((2, CH, D_OUT), jnp.float32),
            pltpu.VMEM((2, CH, D_OUT), jnp.float32),
            pltpu.VMEM((2, CH2, 2, D_W), jnp.int32),
            pltpu.SemaphoreType.DMA,
            pltpu.SemaphoreType.DMA,
            pltpu.SemaphoreType.DMA,
            pltpu.SemaphoreType.DMA,
        ],
    )(_sc_body)
    return k(src, dst, P, Q, E)


def kernel(x, edge_index, edge_attr, W, b):
    W3 = W[2 * D_NODE:, :]
    W3p = jnp.concatenate(
        [W3[:, jnp.asarray(_LO_SEL)], W3[:, jnp.asarray(_HI_SEL)]], axis=1)
    P, Q = _node_proj(x, W, b)
    Ew = _edge_proj(edge_attr.T, W3p)
    return _sc_gather_add(edge_index[0], edge_index[1], P, Q, Ew)
